# baseline (speedup 1.0000x reference)
import functools

import jax
import jax.numpy as jnp
from jax.experimental import pallas as pl
from jax.experimental.pallas import tpu as pltpu

_TAU = 2.0
_GAMMA = 0.7
_IOU_THR = 0.5

_TI = 256
_TJ = 640


def _stiles(m):
    return [(j0, min(_TJ, m - j0)) for j0 in range(0, m, _TJ)]


def _stats_kernel(tb_ref, sb_ref, tl_ref, tm_ref, key_ref, cnt_ref):
    tx1 = tb_ref[0, :, 0:1]
    ty1 = tb_ref[0, :, 1:2]
    tx2 = tb_ref[0, :, 2:3]
    ty2 = tb_ref[0, :, 3:4]
    sx1 = sb_ref[0, :, 0:1]
    sy1 = sb_ref[0, :, 1:2]
    sx2 = sb_ref[0, :, 2:3]
    sy2 = sb_ref[0, :, 3:4]
    area_t = (tx2 - tx1) * (ty2 - ty1)
    area_s = (sx2 - sx1) * (sy2 - sy1)
    wx = jnp.maximum(jnp.minimum(tx2, sx2) - jnp.maximum(tx1, sx1), 0.0)
    wy = jnp.maximum(jnp.minimum(ty2, sy2) - jnp.maximum(ty1, sy1), 0.0)
    inter = wx * wy
    pass0 = inter - area_s * (1.0 / 3.0) >= area_t * (1.0 / 3.0)
    tmv = tm_ref[0] > 0.5
    fa = tmv & jnp.logical_not(pass0)

    zt = tl_ref[0] * (1.0 / _TAU)
    mt = jnp.max(zt, axis=1, keepdims=True)
    st = jnp.sum(jnp.exp(zt - mt), axis=1, keepdims=True)
    fb = tmv & (st < (1.0 / _GAMMA) * (1.0 + 1e-5))

    key_ref[0] = jnp.where(fa, 2.0, 0.0) + jnp.where(fb, 1.0, 0.0)

    na = jnp.sum(jnp.where(fa, 1.0, 0.0))
    nab = jnp.sum(jnp.where(fa & fb, 1.0, 0.0))
    nbo = jnp.sum(jnp.where(fb & jnp.logical_not(fa), 1.0, 0.0))
    cp = jnp.sum(jnp.where(tmv & pass0, 1.0, 0.0))
    lane = jax.lax.broadcasted_iota(jnp.int32, cnt_ref.shape, 2)
    cnt_ref[...] = jnp.where(
        lane == 0, na,
        jnp.where(lane == 1, nab, jnp.where(lane == 2, nbo, cp)))


def _scan_kernel(idx_s, meta_s, tb_ref, tl_ref, sbt_ref, sl_ref,
                 cnt_ref, sum_ref, tbs, tls, *, m):
    i = pl.program_id(0)
    t = pl.program_id(1)
    base = t * _TI

    @pl.when(t == 0)
    def _():
        cnt_ref[...] = jnp.zeros_like(cnt_ref)
        sum_ref[...] = jnp.zeros_like(sum_ref)

    na = meta_s[i * 4]
    nab = meta_s[i * 4 + 1]
    nbo = meta_s[i * 4 + 2]
    nu = meta_s[i * 4 + 3]
    rows_u = jnp.clip(nu - base, 0, _TI)
    b1 = jnp.clip(nab - base, 0, _TI)
    b2 = (jnp.minimum(base + _TI, na + nbo)
          - jnp.maximum(base, na))
    has_b = (b1 + jnp.maximum(b2, 0)) > 0

    def stile(j0, tj):
        sx1 = sbt_ref[0, 0:1, j0:j0 + tj]
        sy1 = sbt_ref[0, 1:2, j0:j0 + tj]
        sx2 = sbt_ref[0, 2:3, j0:j0 + tj]
        sy2 = sbt_ref[0, 3:4, j0:j0 + tj]
        return sx1, sy1, sx2, sy2, (sx2 - sx1) * (sy2 - sy1)

    def gather_u(r, c):
        g = idx_s[i * m + base + r]
        tbs[pl.ds(r, 1), :] = tb_ref[0, pl.ds(g, 1), :]
        tls[pl.ds(r, 1), :] = tl_ref[0, pl.ds(g, 1), :]
        return c

    jax.lax.fori_loop(0, rows_u, gather_u, 0)

    tx1 = tbs[:, 0:1]
    ty1 = tbs[:, 1:2]
    tx2 = tbs[:, 2:3]
    ty2 = tbs[:, 3:4]
    area_t = (tx2 - tx1) * (ty2 - ty1)

    def scan_h():
        hmax = jnp.full((_TI, 1), -jnp.inf, jnp.float32)
        for j0, tj in _stiles(m):
            sx1, sy1, sx2, sy2, area_s = stile(j0, tj)
            wx = jnp.maximum(jnp.minimum(tx2, sx2) - jnp.maximum(tx1, sx1),
                             0.0)
            wy = jnp.maximum(jnp.minimum(ty2, sy2) - jnp.maximum(ty1, sy1),
                             0.0)
            h = wx * wy - area_s * (1.0 / 3.0)
            hmax = jnp.maximum(hmax, jnp.max(h, axis=1, keepdims=True))
        return hmax

    hmax = jax.lax.cond(
        rows_u > 0, scan_h,
        lambda: jnp.full((_TI, 1), -jnp.inf, jnp.float32))
    kept = hmax >= area_t * (1.0 / 3.0)

    pos = base + jax.lax.broadcasted_iota(jnp.int32, (_TI, 1), 0)
    valid_a = pos < na
    valid_b = (pos < nab) | ((pos >= na) & (pos < na + nbo))

    cnt_add = jnp.sum(jnp.where(kept & valid_a, 1.0, 0.0))

    def scan_q():
        zt = tls[...] * (1.0 / _TAU)
        mt = jnp.max(zt, axis=1, keepdims=True)
        et = jnp.exp(zt - mt)
        st = jnp.sum(et, axis=1, keepdims=True)
        lse_t = mt + jnp.log(st)
        p_t = et / st
        ent = jnp.sum(p_t * zt, axis=1, keepdims=True) - lse_t
        conf = jnp.max(p_t, axis=1, keepdims=True)
        w = jnp.clip((conf - _GAMMA) / (1.0 - _GAMMA), 0.0, 1.0)
        pts_ext = jnp.concatenate(
            [p_t * (-1.0 / _TAU), jnp.ones((_TI, 1), jnp.float32)], axis=1)

        def iou_tile(j0, tj):
            sx1, sy1, sx2, sy2, area_s = stile(j0, tj)
            wx = jnp.maximum(jnp.minimum(tx2, sx2) - jnp.maximum(tx1, sx1),
                             0.0)
            wy = jnp.maximum(jnp.minimum(ty2, sy2) - jnp.maximum(ty1, sy1),
                             0.0)
            inter = wx * wy
            union = area_t + area_s - inter
            return inter / jnp.maximum(union, 1e-12)

        best = jnp.full((_TI, 1), -jnp.inf, jnp.float32)
        for j0, tj in _stiles(m):
            best = jnp.maximum(
                best, jnp.max(iou_tile(j0, tj), axis=1, keepdims=True))

        qb = jnp.full((_TI, 1), -jnp.inf, jnp.float32)
        for j0, tj in _stiles(m):
            sl_t = sl_ref[0, j0:j0 + tj, :]
            zs = sl_t * (1.0 / _TAU)
            ms = jnp.max(zs, axis=1, keepdims=True)
            lse_s = ms + jnp.log(jnp.sum(jnp.exp(zs - ms), axis=1,
                                         keepdims=True))
            sl_ext = jnp.concatenate([sl_t, lse_s], axis=1)
            q = jax.lax.dot_general(
                pts_ext, sl_ext,
                dimension_numbers=(((1,), (1,)), ((), ())),
                preferred_element_type=jnp.float32)
            qsel = jnp.max(jnp.where(iou_tile(j0, tj) == best, q, -jnp.inf),
                           axis=1, keepdims=True)
            qb = jnp.maximum(qb, qsel)

        kl = ent + qb
        terms = w * (_TAU * _TAU) * kl
        return jnp.sum(jnp.where(kept & valid_b, terms, 0.0))

    sum_add = jax.lax.cond(has_b, scan_q, lambda: 0.0)

    cnt_ref[...] += jnp.full(cnt_ref.shape, cnt_add, jnp.float32)
    sum_ref[...] += jnp.full(sum_ref.shape, sum_add, jnp.float32)


def kernel(t_boxes, t_logits, t_valid, s_boxes, s_logits, s_valid):
    B, M, C = t_logits.shape
    dt = jnp.float32

    tbp = t_boxes.astype(dt)
    sbp = s_boxes.astype(dt)
    tlp = t_logits.astype(dt)
    slp = s_logits.astype(dt)
    tmf = t_valid.astype(dt)[..., None]
    sbt = sbp.transpose(0, 2, 1)

    key, counts = pl.pallas_call(
        _stats_kernel,
        grid=(B,),
        in_specs=[
            pl.BlockSpec((1, M, 4), lambda i: (i, 0, 0)),
            pl.BlockSpec((1, M, 4), lambda i: (i, 0, 0)),
            pl.BlockSpec((1, M, C), lambda i: (i, 0, 0)),
            pl.BlockSpec((1, M, 1), lambda i: (i, 0, 0)),
        ],
        out_specs=[
            pl.BlockSpec((1, M, 1), lambda i: (i, 0, 0)),
            pl.BlockSpec((1, 8, 128), lambda i: (i, 0, 0)),
        ],
        out_shape=[
            jax.ShapeDtypeStruct((B, M, 1), dt),
            jax.ShapeDtypeStruct((B, 8, 128), dt),
        ],
        compiler_params=pltpu.CompilerParams(
            dimension_semantics=("parallel",)),
    )(tbp, sbp, tlp, tmf)

    idx = jnp.argsort(-key[:, :, 0], axis=1, stable=True)
    idx = idx.astype(jnp.int32).reshape(-1)
    na = counts[:, 0, 0].astype(jnp.int32)
    nab = counts[:, 0, 1].astype(jnp.int32)
    nbo = counts[:, 0, 2].astype(jnp.int32)
    cp = counts[:, 0, 3]
    meta = jnp.stack([na, nab, nbo, na + nbo], axis=1).reshape(-1)

    nt = (M + _TI - 1) // _TI
    grid_spec = pltpu.PrefetchScalarGridSpec(
        num_scalar_prefetch=2,
        grid=(B, nt),
        in_specs=[
            pl.BlockSpec((1, M, 4), lambda i, t, *_: (i, 0, 0)),
            pl.BlockSpec((1, M, C), lambda i, t, *_: (i, 0, 0)),
            pl.BlockSpec((1, 4, M), lambda i, t, *_: (i, 0, 0)),
            pl.BlockSpec((1, M, C), lambda i, t, *_: (i, 0, 0)),
        ],
        out_specs=[
            pl.BlockSpec((1, 8, 128), lambda i, t, *_: (i, 0, 0)),
            pl.BlockSpec((1, 8, 128), lambda i, t, *_: (i, 0, 0)),
        ],
        scratch_shapes=[
            pltpu.VMEM((_TI, 4), dt),
            pltpu.VMEM((_TI, C), dt),
        ],
    )
    cnts, sums = pl.pallas_call(
        functools.partial(_scan_kernel, m=M),
        grid_spec=grid_spec,
        out_shape=[
            jax.ShapeDtypeStruct((B, 8, 128), dt),
            jax.ShapeDtypeStruct((B, 8, 128), dt),
        ],
        compiler_params=pltpu.CompilerParams(
            dimension_semantics=("parallel", "arbitrary")),
    )(idx, meta, tbp, tlp, sbt, slp)

    s = sums[:, 0, 0]
    n = cp + cnts[:, 0, 0]
    has = n > 0
    loss_i = jnp.where(has, s / jnp.maximum(n, 1.0), 0.0)
    loss_sum = jnp.sum(loss_i)
    denom = jnp.sum(has.astype(dt))
    return jnp.where(denom == 0, loss_sum, loss_sum / jnp.maximum(denom, 1.0))


# speedup vs baseline: 1.0377x; 1.0377x over previous
import functools

import jax
import jax.numpy as jnp
from jax.experimental import pallas as pl
from jax.experimental.pallas import tpu as pltpu

_TAU = 2.0
_GAMMA = 0.7
_IOU_THR = 0.5

_TI = 256
_TJ = 640


def _stiles(m):
    return [(j0, min(_TJ, m - j0)) for j0 in range(0, m, _TJ)]


def _stats_kernel(tb_ref, sb_ref, tl_ref, tm_ref, key_ref, cnt_ref):
    tx1 = tb_ref[0, :, 0:1]
    ty1 = tb_ref[0, :, 1:2]
    tx2 = tb_ref[0, :, 2:3]
    ty2 = tb_ref[0, :, 3:4]
    sx1 = sb_ref[0, :, 0:1]
    sy1 = sb_ref[0, :, 1:2]
    sx2 = sb_ref[0, :, 2:3]
    sy2 = sb_ref[0, :, 3:4]
    area_t = (tx2 - tx1) * (ty2 - ty1)
    area_s = (sx2 - sx1) * (sy2 - sy1)
    wx = jnp.maximum(jnp.minimum(tx2, sx2) - jnp.maximum(tx1, sx1), 0.0)
    wy = jnp.maximum(jnp.minimum(ty2, sy2) - jnp.maximum(ty1, sy1), 0.0)
    inter = wx * wy
    pass0 = inter - area_s * (1.0 / 3.0) >= area_t * (1.0 / 3.0)
    tmv = tm_ref[0] > 0.5
    fa = tmv & jnp.logical_not(pass0)

    zt = tl_ref[0] * (1.0 / _TAU)
    mt = jnp.max(zt, axis=1, keepdims=True)
    st = jnp.sum(jnp.exp(zt - mt), axis=1, keepdims=True)
    fb = tmv & (st < (1.0 / _GAMMA) * (1.0 + 1e-5))

    key_ref[0] = jnp.where(fa, 2.0, 0.0) + jnp.where(fb, 1.0, 0.0)

    na = jnp.sum(jnp.where(fa, 1.0, 0.0))
    nab = jnp.sum(jnp.where(fa & fb, 1.0, 0.0))
    nbo = jnp.sum(jnp.where(fb & jnp.logical_not(fa), 1.0, 0.0))
    cp = jnp.sum(jnp.where(tmv & pass0, 1.0, 0.0))
    lane = jax.lax.broadcasted_iota(jnp.int32, cnt_ref.shape, 2)
    cnt_ref[...] = jnp.where(
        lane == 0, na,
        jnp.where(lane == 1, nab, jnp.where(lane == 2, nbo, cp)))


def _scan_kernel(idx_s, meta_s, tb_ref, tl_ref, sbt_ref, sl_ref,
                 cnt_ref, sum_ref, tbs, tls, *, m):
    i = pl.program_id(0)
    t = pl.program_id(1)
    base = t * _TI

    @pl.when(t == 0)
    def _():
        cnt_ref[...] = jnp.zeros_like(cnt_ref)
        sum_ref[...] = jnp.zeros_like(sum_ref)

    na = meta_s[i * 4]
    nab = meta_s[i * 4 + 1]
    nbo = meta_s[i * 4 + 2]
    nu = meta_s[i * 4 + 3]
    rows_u = jnp.clip(nu - base, 0, _TI)
    b1 = jnp.clip(nab - base, 0, _TI)
    b2 = (jnp.minimum(base + _TI, na + nbo)
          - jnp.maximum(base, na))
    has_b = (b1 + jnp.maximum(b2, 0)) > 0

    def stile(j0, tj):
        sx1 = sbt_ref[0, 0:1, j0:j0 + tj]
        sy1 = sbt_ref[0, 1:2, j0:j0 + tj]
        sx2 = sbt_ref[0, 2:3, j0:j0 + tj]
        sy2 = sbt_ref[0, 3:4, j0:j0 + tj]
        return sx1, sy1, sx2, sy2, (sx2 - sx1) * (sy2 - sy1)

    def gather_u(r, c):
        g = idx_s[i * m + base + r]
        tbs[pl.ds(r, 1), :] = tb_ref[0, pl.ds(g, 1), :]
        tls[pl.ds(r, 1), :] = tl_ref[0, pl.ds(g, 1), :]
        return c

    jax.lax.fori_loop(0, rows_u, gather_u, 0)

    tx1 = tbs[:, 0:1]
    ty1 = tbs[:, 1:2]
    tx2 = tbs[:, 2:3]
    ty2 = tbs[:, 3:4]
    area_t = (tx2 - tx1) * (ty2 - ty1)

    def scan_h():
        hmax = jnp.full((_TI, 1), -jnp.inf, jnp.float32)
        for j0, tj in _stiles(m):
            sx1, sy1, sx2, sy2, area_s = stile(j0, tj)
            wx = jnp.maximum(jnp.minimum(tx2, sx2) - jnp.maximum(tx1, sx1),
                             0.0)
            wy = jnp.maximum(jnp.minimum(ty2, sy2) - jnp.maximum(ty1, sy1),
                             0.0)
            h = wx * wy - area_s * (1.0 / 3.0)
            hmax = jnp.maximum(hmax, jnp.max(h, axis=1, keepdims=True))
        return hmax

    hmax = jax.lax.cond(
        rows_u > 0, scan_h,
        lambda: jnp.full((_TI, 1), -jnp.inf, jnp.float32))
    kept = hmax >= area_t * (1.0 / 3.0)

    pos = base + jax.lax.broadcasted_iota(jnp.int32, (_TI, 1), 0)
    valid_a = pos < na
    valid_b = (pos < nab) | ((pos >= na) & (pos < na + nbo))

    cnt_add = jnp.sum(jnp.where(kept & valid_a, 1.0, 0.0))

    def scan_q():
        zt = tls[...] * (1.0 / _TAU)
        mt = jnp.max(zt, axis=1, keepdims=True)
        et = jnp.exp(zt - mt)
        st = jnp.sum(et, axis=1, keepdims=True)
        lse_t = mt + jnp.log(st)
        p_t = et / st
        ent = jnp.sum(p_t * zt, axis=1, keepdims=True) - lse_t
        conf = jnp.max(p_t, axis=1, keepdims=True)
        w = jnp.clip((conf - _GAMMA) / (1.0 - _GAMMA), 0.0, 1.0)
        pts_ext = jnp.concatenate(
            [p_t * (-1.0 / _TAU), jnp.ones((_TI, 1), jnp.float32)], axis=1)

        def iou_tile(j0, tj):
            sx1, sy1, sx2, sy2, area_s = stile(j0, tj)
            wx = jnp.maximum(jnp.minimum(tx2, sx2) - jnp.maximum(tx1, sx1),
                             0.0)
            wy = jnp.maximum(jnp.minimum(ty2, sy2) - jnp.maximum(ty1, sy1),
                             0.0)
            inter = wx * wy
            union = area_t + area_s - inter
            return inter / jnp.maximum(union, 1e-12)

        best = jnp.full((_TI, 1), -jnp.inf, jnp.float32)
        for j0, tj in _stiles(m):
            best = jnp.maximum(
                best, jnp.max(iou_tile(j0, tj), axis=1, keepdims=True))

        qb = jnp.full((_TI, 1), -jnp.inf, jnp.float32)
        for j0, tj in _stiles(m):
            sl_t = sl_ref[0, j0:j0 + tj, :]
            zs = sl_t * (1.0 / _TAU)
            ms = jnp.max(zs, axis=1, keepdims=True)
            lse_s = ms + jnp.log(jnp.sum(jnp.exp(zs - ms), axis=1,
                                         keepdims=True))
            sl_ext = jnp.concatenate([sl_t, lse_s], axis=1)
            q = jax.lax.dot_general(
                pts_ext, sl_ext,
                dimension_numbers=(((1,), (1,)), ((), ())),
                preferred_element_type=jnp.float32)
            qsel = jnp.max(jnp.where(iou_tile(j0, tj) == best, q, -jnp.inf),
                           axis=1, keepdims=True)
            qb = jnp.maximum(qb, qsel)

        kl = ent + qb
        terms = w * (_TAU * _TAU) * kl
        return jnp.sum(jnp.where(kept & valid_b, terms, 0.0))

    sum_add = jax.lax.cond(has_b, scan_q, lambda: 0.0)

    cnt_ref[...] += jnp.full(cnt_ref.shape, cnt_add, jnp.float32)
    sum_ref[...] += jnp.full(sum_ref.shape, sum_add, jnp.float32)


def kernel(t_boxes, t_logits, t_valid, s_boxes, s_logits, s_valid):
    B, M, C = t_logits.shape
    dt = jnp.float32

    tbp = t_boxes.astype(dt)
    sbp = s_boxes.astype(dt)
    tlp = t_logits.astype(dt)
    slp = s_logits.astype(dt)
    tmf = t_valid.astype(dt)[..., None]
    sbt = sbp.transpose(0, 2, 1)

    key, counts = pl.pallas_call(
        _stats_kernel,
        grid=(B,),
        in_specs=[
            pl.BlockSpec((1, M, 4), lambda i: (i, 0, 0)),
            pl.BlockSpec((1, M, 4), lambda i: (i, 0, 0)),
            pl.BlockSpec((1, M, C), lambda i: (i, 0, 0)),
            pl.BlockSpec((1, M, 1), lambda i: (i, 0, 0)),
        ],
        out_specs=[
            pl.BlockSpec((1, M, 1), lambda i: (i, 0, 0)),
            pl.BlockSpec((1, 8, 128), lambda i: (i, 0, 0)),
        ],
        out_shape=[
            jax.ShapeDtypeStruct((B, M, 1), dt),
            jax.ShapeDtypeStruct((B, 8, 128), dt),
        ],
        compiler_params=pltpu.CompilerParams(
            dimension_semantics=("parallel",)),
    )(tbp, sbp, tlp, tmf)

    idx = jnp.argsort(-key[:, :, 0], axis=1, stable=False)
    idx = idx.astype(jnp.int32).reshape(-1)
    na = counts[:, 0, 0].astype(jnp.int32)
    nab = counts[:, 0, 1].astype(jnp.int32)
    nbo = counts[:, 0, 2].astype(jnp.int32)
    cp = counts[:, 0, 3]
    meta = jnp.stack([na, nab, nbo, na + nbo], axis=1).reshape(-1)

    nt = (M + _TI - 1) // _TI
    grid_spec = pltpu.PrefetchScalarGridSpec(
        num_scalar_prefetch=2,
        grid=(B, nt),
        in_specs=[
            pl.BlockSpec((1, M, 4), lambda i, t, *_: (i, 0, 0)),
            pl.BlockSpec((1, M, C), lambda i, t, *_: (i, 0, 0)),
            pl.BlockSpec((1, 4, M), lambda i, t, *_: (i, 0, 0)),
            pl.BlockSpec((1, M, C), lambda i, t, *_: (i, 0, 0)),
        ],
        out_specs=[
            pl.BlockSpec((1, 8, 128), lambda i, t, *_: (i, 0, 0)),
            pl.BlockSpec((1, 8, 128), lambda i, t, *_: (i, 0, 0)),
        ],
        scratch_shapes=[
            pltpu.VMEM((_TI, 4), dt),
            pltpu.VMEM((_TI, C), dt),
        ],
    )
    cnts, sums = pl.pallas_call(
        functools.partial(_scan_kernel, m=M),
        grid_spec=grid_spec,
        out_shape=[
            jax.ShapeDtypeStruct((B, 8, 128), dt),
            jax.ShapeDtypeStruct((B, 8, 128), dt),
        ],
        compiler_params=pltpu.CompilerParams(
            dimension_semantics=("parallel", "arbitrary")),
    )(idx, meta, tbp, tlp, sbt, slp)

    s = sums[:, 0, 0]
    n = cp + cnts[:, 0, 0]
    has = n > 0
    loss_i = jnp.where(has, s / jnp.maximum(n, 1.0), 0.0)
    loss_sum = jnp.sum(loss_i)
    denom = jnp.sum(has.astype(dt))
    return jnp.where(denom == 0, loss_sum, loss_sum / jnp.maximum(denom, 1.0))
